# baseline (speedup 1.0000x reference)
import jax
import jax.numpy as jnp
from jax.experimental import pallas as pl
from jax.experimental.pallas import tpu as pltpu

_N_BLK = 2048


def _mm_kernel(u_ref, i_ref, o_ref):
    o_ref[...] = jax.lax.dot_general(
        u_ref[...],
        i_ref[...],
        dimension_numbers=(((1,), (1,)), ((), ())),
        preferred_element_type=jnp.float32,
    )


def kernel(u_g_embeddings, i_g_embeddings):
    M, K = u_g_embeddings.shape
    N = i_g_embeddings.shape[0]
    return pl.pallas_call(
        _mm_kernel,
        grid=(pl.cdiv(N, _N_BLK),),
        in_specs=[
            pl.BlockSpec((M, K), lambda j: (0, 0)),
            pl.BlockSpec((_N_BLK, K), lambda j: (j, 0)),
        ],
        out_specs=pl.BlockSpec((M, _N_BLK), lambda j: (0, j)),
        out_shape=jax.ShapeDtypeStruct((M, N), jnp.float32),
        compiler_params=pltpu.CompilerParams(
            dimension_semantics=("parallel",),
        ),
    )(u_g_embeddings, i_g_embeddings)


# speedup vs baseline: 1.0905x; 1.0905x over previous
import jax
import jax.numpy as jnp
from jax.experimental import pallas as pl
from jax.experimental.pallas import tpu as pltpu

_M_BLK = 32


def _mm_kernel(u_ref, it_ref, o_ref):
    o_ref[...] = jnp.dot(u_ref[...], it_ref[...],
                         preferred_element_type=jnp.float32)


def kernel(u_g_embeddings, i_g_embeddings):
    M, K = u_g_embeddings.shape
    N = i_g_embeddings.shape[0]
    it = i_g_embeddings.T
    return pl.pallas_call(
        _mm_kernel,
        grid=(M // _M_BLK,),
        in_specs=[
            pl.BlockSpec((_M_BLK, K), lambda m: (m, 0)),
            pl.BlockSpec((K, N), lambda m: (0, 0)),
        ],
        out_specs=pl.BlockSpec((_M_BLK, N), lambda m: (m, 0)),
        out_shape=jax.ShapeDtypeStruct((M, N), jnp.float32),
        compiler_params=pltpu.CompilerParams(
            dimension_semantics=("parallel",),
        ),
    )(u_g_embeddings, it)


# speedup vs baseline: 4.1472x; 3.8031x over previous
import jax
import jax.numpy as jnp
from jax.experimental import pallas as pl
from jax.experimental.pallas import tpu as pltpu

_N_BLK = 2048


def _mm_kernel(it_ref, ut_ref, o_ref):
    o_ref[...] = jax.lax.dot_general(
        it_ref[...],
        ut_ref[...],
        dimension_numbers=(((0,), (0,)), ((), ())),
        preferred_element_type=jnp.float32,
    )


def kernel(u_g_embeddings, i_g_embeddings):
    M, K = u_g_embeddings.shape
    N = i_g_embeddings.shape[0]
    ut = u_g_embeddings.T
    it = i_g_embeddings.T
    out_t = pl.pallas_call(
        _mm_kernel,
        grid=(pl.cdiv(N, _N_BLK),),
        in_specs=[
            pl.BlockSpec((K, _N_BLK), lambda j: (0, j)),
            pl.BlockSpec((K, M), lambda j: (0, 0)),
        ],
        out_specs=pl.BlockSpec((_N_BLK, M), lambda j: (j, 0)),
        out_shape=jax.ShapeDtypeStruct((N, M), jnp.float32),
        compiler_params=pltpu.CompilerParams(
            dimension_semantics=("parallel",),
        ),
    )(it, ut)
    return out_t.T
